# baseline (speedup 1.0000x reference)
import functools

import jax
import jax.numpy as jnp
from jax import lax
from jax.experimental import pallas as pl
from jax.experimental.pallas import tpu as pltpu
from jax.experimental.pallas import tpu_sc as plsc

_B = 4096
_L = 20
_D = 4
_LANES = 16
_NC = 2
_NS = 16
_NW = _NC * _NS
_BPW = _B // _NW
_G = _BPW // _LANES
_GI = 4
_NTOK = 128
_TBL_ROWS = 32

_SH = 32 - _L
_ONE = 1 << _SH
_VP0 = -(1 << _SH)
_WPW = _BPW * _L
_A_OFF = 0
_B_OFF = _B * _L
_T_OFF = 2 * _B * _L


def _splat(v):
    return jnp.full((_LANES,), v, jnp.int32)


def _sc_body(cat_hbm, out_hbm, a_v, b_v, tbl_v, peq_v, out_v, sa, sb, st):
    wid = lax.axis_index("s") * _NC + lax.axis_index("c")
    cpa = pltpu.async_copy(cat_hbm.at[pl.ds(_A_OFF + wid * _WPW, _WPW)], a_v, sa)
    cpb = pltpu.async_copy(cat_hbm.at[pl.ds(_B_OFF + wid * _WPW, _WPW)], b_v, sb)
    cpt = pltpu.async_copy(cat_hbm.at[pl.ds(_T_OFF, _TBL_ROWS * _D)], tbl_v, st)
    lane = lax.broadcasted_iota(jnp.int32, (_LANES,), 0)
    lane_l = lane * _L

    def zero_body(i, carry):
        off = pl.multiple_of(i * 256, _LANES)
        for t in range(16):
            peq_v[pl.ds(off + t * _LANES, _LANES)] = _splat(0)
        return carry

    lax.fori_loop(0, _GI * _NTOK * _LANES // 256, zero_body, 0)
    cpa.wait()
    cpb.wait()
    cpt.wait()

    for half in range(_G // _GI):
        groups = [half * _GI + t for t in range(_GI)]
        pbases = [t * _NTOK * _LANES + lane for t in range(_GI)]

        def build_body(j, carry, groups=groups, pbases=pbases):
            bit = jnp.broadcast_to(jnp.int32(1) << (j + _SH), (_LANES,))
            for t, g in enumerate(groups):
                off = pl.multiple_of(g * _L * _LANES + j * _LANES, _LANES)
                bj = b_v[pl.ds(off, _LANES)]
                plsc.addupdate_scatter(peq_v, [pbases[t] + (bj << 4)], bit)
            return carry

        lax.fori_loop(0, _L, build_body, 0)

        def i_body(i, carry, groups=groups, pbases=pbases):
            vps, vns, scs = carry
            nvps, nvns, nscs = [], [], []
            for t, g in enumerate(groups):
                vp, vn, sc = vps[t], vns[t], scs[t]
                off = pl.multiple_of(g * _L * _LANES + i * _LANES, _LANES)
                ai = a_v[pl.ds(off, _LANES)]
                peq = plsc.load_gather(peq_v, [pbases[t] + (ai << 4)])
                x = peq | vn
                d0 = ((vp + (x & vp)) ^ vp) | x
                hn = vp & d0
                hp = vn | ((vp | d0) ^ -1)
                sc = sc - (hp >> 31) + (hn >> 31)
                xs = (hp << 1) | _ONE
                nvns.append(xs & d0)
                nvps.append((hn << 1) | ((xs | d0) ^ -1))
                nscs.append(sc)
            return (tuple(nvps), tuple(nvns), tuple(nscs))

        init = (tuple(_splat(_VP0) for _ in groups),
                tuple(_splat(0) for _ in groups),
                tuple(_splat(_L) for _ in groups))
        _, _, scores = lax.fori_loop(0, _L, i_body, init, unroll=2)

        if half + 1 < _G // _GI:
            def clear_body(j, carry, groups=groups, pbases=pbases):
                for t, g in enumerate(groups):
                    off = pl.multiple_of(g * _L * _LANES + j * _LANES, _LANES)
                    bj = b_v[pl.ds(off, _LANES)]
                    plsc.store_scatter(peq_v, [pbases[t] + (bj << 4)], _splat(0))
                return carry

            lax.fori_loop(0, _L, clear_body, 0)

        for t, g in enumerate(groups):
            dist = jnp.minimum(scores[t], _TBL_ROWS - 1)
            base = dist * _D
            orow = g * _LANES + lane
            for c in range(_D):
                col = plsc.load_gather(tbl_v, [base + c])
                plsc.store_scatter(
                    out_v, [orow, _splat(c)], plsc.bitcast(col, jnp.float32))

    pltpu.sync_copy(out_v, out_hbm.at[pl.ds(wid * _BPW, _BPW)])


_sc_call = functools.partial(
    pl.kernel,
    mesh=plsc.VectorSubcoreMesh(core_axis_name="c", subcore_axis_name="s"),
    out_type=jax.ShapeDtypeStruct((_B, _D), jnp.float32),
    compiler_params=pltpu.CompilerParams(
        needs_layout_passes=False,
        disable_bounds_checks=True,
        disable_semaphore_checks=True,
        skip_device_barrier=True,
    ),
    scratch_types=[
        pltpu.VMEM((_WPW,), jnp.int32),
        pltpu.VMEM((_WPW,), jnp.int32),
        pltpu.VMEM((_TBL_ROWS * _D,), jnp.int32),
        pltpu.VMEM((_GI * _NTOK * _LANES,), jnp.int32),
        pltpu.VMEM((_BPW, _D), jnp.float32),
        pltpu.SemaphoreType.DMA,
        pltpu.SemaphoreType.DMA,
        pltpu.SemaphoreType.DMA,
    ],
)(_sc_body)


def kernel(input1, input2, embedding_table):
    packed = (jnp.stack([input1, input2])
              .reshape(2, _NW, _G, _LANES, _L)
              .transpose(0, 1, 2, 4, 3))
    cat = jnp.concatenate([
        packed.reshape(-1),
        jax.lax.bitcast_convert_type(
            embedding_table[:_TBL_ROWS], jnp.int32).reshape(-1),
    ])
    return _sc_call(cat)


# speedup vs baseline: 1.2363x; 1.2363x over previous
import functools

import jax
import jax.numpy as jnp
from jax import lax
from jax.experimental import pallas as pl
from jax.experimental.pallas import tpu as pltpu
from jax.experimental.pallas import tpu_sc as plsc

_B = 4096
_L = 20
_D = 4
_LANES = 16
_NC = 2
_NS = 16
_NW = _NC * _NS
_BPW = _B // _NW
_G = _BPW // _LANES
_GI = 4
_NTOK = 128
_TBL_ROWS = 32

_SH = 32 - _L
_ONE = 1 << _SH
_VP0 = -(1 << _SH)
_WPW = _BPW * _L
_A_OFF = 0
_B_OFF = _B * _L
_T_OFF = 2 * _B * _L


def _splat(v):
    return jnp.full((_LANES,), v, jnp.int32)


def _sc_body(cat_hbm, out_hbm, a_v, b_v, tbl_v, peq_v, out_v, sa, sb, st):
    wid = lax.axis_index("s") * _NC + lax.axis_index("c")
    cpa = pltpu.async_copy(cat_hbm.at[pl.ds(_A_OFF + wid * _WPW, _WPW)], a_v, sa)
    cpb = pltpu.async_copy(cat_hbm.at[pl.ds(_B_OFF + wid * _WPW, _WPW)], b_v, sb)
    cpt = pltpu.async_copy(cat_hbm.at[pl.ds(_T_OFF, _TBL_ROWS * _D)], tbl_v, st)
    lane = lax.broadcasted_iota(jnp.int32, (_LANES,), 0)
    lane_l = lane * _L

    def zero_body(i, carry):
        off = pl.multiple_of(i * 256, _LANES)
        for t in range(16):
            peq_v[pl.ds(off + t * _LANES, _LANES)] = _splat(0)
        return carry

    lax.fori_loop(0, _GI * _NTOK * _LANES // 256, zero_body, 0)
    cpa.wait()
    cpb.wait()
    cpt.wait()

    for half in range(_G // _GI):
        groups = [half * _GI + t for t in range(_GI)]
        pbases = [t * _NTOK * _LANES + lane for t in range(_GI)]

        def build_body(j, idxs, pbases=pbases):
            bit = jnp.broadcast_to(jnp.int32(1) << (j + _SH), (_LANES,))
            nidxs = []
            for t in range(_GI):
                bj = plsc.load_gather(b_v, [idxs[t]])
                plsc.addupdate_scatter(peq_v, [pbases[t] + (bj << 4)], bit)
                nidxs.append(idxs[t] + 1)
            return tuple(nidxs)

        lax.fori_loop(0, _L, build_body,
                      tuple(lane_l + g * _LANES * _L for g in groups))

        def i_body(i, carry, pbases=pbases):
            del i
            vps, vns, scs, idxs = carry
            nvps, nvns, nscs, nidxs = [], [], [], []
            for t in range(_GI):
                vp, vn, sc, idxa = vps[t], vns[t], scs[t], idxs[t]
                ai = plsc.load_gather(a_v, [idxa])
                peq = plsc.load_gather(peq_v, [pbases[t] + (ai << 4)])
                x = peq | vn
                d0 = ((vp + (x & vp)) ^ vp) | x
                hn = vp & d0
                hp = vn | ((vp | d0) ^ -1)
                sc = sc - (hp >> 31) + (hn >> 31)
                xs = (hp << 1) | _ONE
                nvns.append(xs & d0)
                nvps.append((hn << 1) | ((xs | d0) ^ -1))
                nscs.append(sc)
                nidxs.append(idxa + 1)
            return (tuple(nvps), tuple(nvns), tuple(nscs), tuple(nidxs))

        init = (tuple(_splat(_VP0) for _ in groups),
                tuple(_splat(0) for _ in groups),
                tuple(_splat(_L) for _ in groups),
                tuple(lane_l + g * _LANES * _L for g in groups))
        _, _, scores, _ = lax.fori_loop(0, _L, i_body, init, unroll=2)

        if half + 1 < _G // _GI:
            def clear_body(j, idxs, pbases=pbases):
                del j
                nidxs = []
                for t in range(_GI):
                    bj = plsc.load_gather(b_v, [idxs[t]])
                    plsc.store_scatter(peq_v, [pbases[t] + (bj << 4)], _splat(0))
                    nidxs.append(idxs[t] + 1)
                return tuple(nidxs)

            lax.fori_loop(0, _L, clear_body,
                          tuple(lane_l + g * _LANES * _L for g in groups))

        for t, g in enumerate(groups):
            dist = jnp.minimum(scores[t], _TBL_ROWS - 1)
            base = dist * _D
            orow = g * _LANES + lane
            for c in range(_D):
                col = plsc.load_gather(tbl_v, [base + c])
                plsc.store_scatter(
                    out_v, [orow, _splat(c)], plsc.bitcast(col, jnp.float32))

    pltpu.sync_copy(out_v, out_hbm.at[pl.ds(wid * _BPW, _BPW)])


_sc_call = functools.partial(
    pl.kernel,
    mesh=plsc.VectorSubcoreMesh(core_axis_name="c", subcore_axis_name="s"),
    out_type=jax.ShapeDtypeStruct((_B, _D), jnp.float32),
    compiler_params=pltpu.CompilerParams(
        needs_layout_passes=False,
        disable_bounds_checks=True,
        disable_semaphore_checks=True,
        skip_device_barrier=True,
    ),
    scratch_types=[
        pltpu.VMEM((_WPW,), jnp.int32),
        pltpu.VMEM((_WPW,), jnp.int32),
        pltpu.VMEM((_TBL_ROWS * _D,), jnp.int32),
        pltpu.VMEM((_GI * _NTOK * _LANES,), jnp.int32),
        pltpu.VMEM((_BPW, _D), jnp.float32),
        pltpu.SemaphoreType.DMA,
        pltpu.SemaphoreType.DMA,
        pltpu.SemaphoreType.DMA,
    ],
)(_sc_body)


def kernel(input1, input2, embedding_table):
    cat = jnp.concatenate([
        input1.reshape(-1),
        input2.reshape(-1),
        jax.lax.bitcast_convert_type(
            embedding_table[:_TBL_ROWS], jnp.int32).reshape(-1),
    ])
    return _sc_call(cat)
